# baseline (speedup 1.0000x reference)
import functools

import jax
import jax.numpy as jnp
from jax import lax
from jax.experimental import pallas as pl
from jax.experimental.pallas import tpu as pltpu
from jax.experimental.pallas import tpu_sc as plsc

ENT = 100000
REL = 1000
D = 128
H = D // 2
B = 16384
LMBDA = 0.001

NC = 2
NS = 16
L = 16
NW = NC * NS
BW = B // NW
C = 64
NCHUNK = BW // C
NBUF = 2


def _pair_body(a_ref, b_ref, out_ref):
    eye = (lax.broadcasted_iota(jnp.int32, (H, H), 0)
           == lax.broadcasted_iota(jnp.int32, (H, H), 1)).astype(jnp.float32)
    dn = (((0,), (0,)), ((), ()))
    left = lax.dot_general(a_ref[...], eye, dn,
                           preferred_element_type=jnp.float32)
    right = lax.dot_general(b_ref[...], eye, dn,
                            preferred_element_type=jnp.float32)
    out_ref[...] = jnp.concatenate([left, right], axis=1)


def _pair_table(at, bt, n_rows, w):
    grid = (n_rows + w - 1) // w
    return pl.pallas_call(
        _pair_body,
        grid=(grid,),
        in_specs=[
            pl.BlockSpec((H, w), lambda i: (0, i)),
            pl.BlockSpec((H, w), lambda i: (0, i)),
        ],
        out_specs=pl.BlockSpec((w, D), lambda i: (i, 0)),
        out_shape=jax.ShapeDtypeStruct((n_rows, D), jnp.float32),
    )(at, bt)


def _sc_body(h_hbm, t_hbm, r_hbm, e12_hbm, e_hbm, r12_hbm, rel_hbm,
             res_out, ssh_out, ssd_out,
             hi_v, ti_v, ri_v, res_v, ss_v, stage_v,
             e12h_v, eh_v, e12t_v, et_v, r12_v, rel_v,
             sem0, sem1):
    wid = lax.axis_index("s") * NC + lax.axis_index("c")
    base = wid * BW

    pltpu.sync_copy(h_hbm.at[pl.ds(base, BW)], hi_v)
    pltpu.sync_copy(t_hbm.at[pl.ds(base, BW)], ti_v)
    pltpu.sync_copy(r_hbm.at[pl.ds(base, BW)], ri_v)

    sems = (sem0, sem1)

    def issue(g, slot):
        cb = g * C
        hi = hi_v.at[pl.ds(cb, C)]
        ti = ti_v.at[pl.ds(cb, C)]
        ri = ri_v.at[pl.ds(cb, C)]
        descs = []
        for tbl, idx, buf in ((e12_hbm, hi, e12h_v), (e_hbm, hi, eh_v),
                              (e12_hbm, ti, e12t_v), (e_hbm, ti, et_v),
                              (r12_hbm, ri, r12_v), (rel_hbm, ri, rel_v)):
            descs.append(pltpu.async_copy(tbl.at[idx], buf.at[slot], sems[slot]))
        return descs

    pend = [None] * NBUF
    pend[0] = issue(0, 0)
    ssh = jnp.zeros((L,), jnp.float32)
    ssd = jnp.zeros((L,), jnp.float32)
    lane = lax.iota(jnp.int32, L)

    for g in range(NCHUNK):
        slot = g % NBUF
        if g + 1 < NCHUNK:
            pend[(g + 1) % NBUF] = issue(g + 1, (g + 1) % NBUF)
        for dsc in pend[slot]:
            dsc.wait()

        def body(c, carry, slot=slot):
            ssh, ssd = carry
            acc = jnp.zeros((L,), jnp.float32)
            for k in range(H // L):
                s1 = pl.ds(k * L, L)
                s2 = pl.ds(H + k * L, L)
                a1h = e12h_v[slot, c, s1]
                a2h = e12h_v[slot, c, s2]
                a1t = e12t_v[slot, c, s1]
                a2t = e12t_v[slot, c, s2]
                v1 = r12_v[slot, c, s1]
                v2 = r12_v[slot, c, s2]
                acc = acc + (a1h * a1t + a2h * a2t) * v1 \
                          + (a1h * a2t - a2h * a1t) * v2
                ssh = ssh + (a1h * a1h + a2h * a2h) + (a1t * a1t + a2t * a2t) \
                          + (v1 * v1 + v2 * v2)
            for k in range(D // L):
                s = pl.ds(k * L, L)
                ah = eh_v[slot, c, s]
                at = et_v[slot, c, s]
                vr = rel_v[slot, c, s]
                acc = acc + ah * at * vr
                ssd = ssd + ah * ah + at * at + vr * vr
            plsc.store_scatter(stage_v, [lane * C + c], acc)
            return (ssh, ssd)

        ssh, ssd = lax.fori_loop(0, C, body, (ssh, ssd), unroll=4)

        for k in range(C // L):
            rv = stage_v[pl.ds(k * L, L)]
            for l in range(1, L):
                rv = rv + stage_v[pl.ds(l * C + k * L, L)]
            res_v[pl.ds(g * C + k * L, L)] = rv

    ss_v[0] = ssh
    ss_v[1] = ssd
    pltpu.sync_copy(res_v, res_out.at[pl.ds(base, BW)])
    pltpu.sync_copy(ss_v.at[0], ssh_out.at[pl.ds(wid * L, L)])
    pltpu.sync_copy(ss_v.at[1], ssd_out.at[pl.ds(wid * L, L)])


_sc_kernel = functools.partial(
    pl.kernel,
    out_type=(
        jax.ShapeDtypeStruct((B,), jnp.float32),
        jax.ShapeDtypeStruct((NW * L,), jnp.float32),
        jax.ShapeDtypeStruct((NW * L,), jnp.float32),
    ),
    mesh=plsc.VectorSubcoreMesh(core_axis_name="c", subcore_axis_name="s"),
    compiler_params=pltpu.CompilerParams(needs_layout_passes=False),
    scratch_types=[
        pltpu.VMEM((BW,), jnp.int32),
        pltpu.VMEM((BW,), jnp.int32),
        pltpu.VMEM((BW,), jnp.int32),
        pltpu.VMEM((BW,), jnp.float32),
        pltpu.VMEM((2, L), jnp.float32),
        pltpu.VMEM((L * C,), jnp.float32),
        pltpu.VMEM((NBUF, C, D), jnp.float32),
        pltpu.VMEM((NBUF, C, D), jnp.float32),
        pltpu.VMEM((NBUF, C, D), jnp.float32),
        pltpu.VMEM((NBUF, C, D), jnp.float32),
        pltpu.VMEM((NBUF, C, D), jnp.float32),
        pltpu.VMEM((NBUF, C, D), jnp.float32),
        pltpu.SemaphoreType.DMA,
        pltpu.SemaphoreType.DMA,
    ],
)(_sc_body)


def _finish_body(res_ref, y_ref, ssh_ref, ssd_ref, out_ref):
    x = -(y_ref[...] * res_ref[...])
    sp = jnp.maximum(x, 0.0) + jnp.log(1.0 + jnp.exp(-jnp.abs(x)))
    reg = jnp.sum(ssh_ref[...]) / (B * H) + jnp.sum(ssd_ref[...]) / (B * D)
    loss = jnp.sum(sp) / B + LMBDA * reg
    out_ref[...] = jnp.full((1, 1), loss, jnp.float32)


def kernel(h, t, r, y, E1, E2, E, R1, R2, R):
    h = h.astype(jnp.int32)
    t = t.astype(jnp.int32)
    r = r.astype(jnp.int32)
    e12 = _pair_table(E1.T, E2.T, ENT, 12800)
    r12 = _pair_table(R1.T, R2.T, REL, 1024)
    res, ssh, ssd = _sc_kernel(h, t, r, e12, E, r12, R)
    loss = pl.pallas_call(
        _finish_body,
        out_shape=jax.ShapeDtypeStruct((1, 1), jnp.float32),
    )(res.reshape(128, 128), y.reshape(128, 128),
      ssh.reshape(4, 128), ssd.reshape(4, 128))
    return loss[0, 0]


# speedup vs baseline: 1.0101x; 1.0101x over previous
import functools

import jax
import jax.numpy as jnp
from jax import lax
from jax.experimental import pallas as pl
from jax.experimental.pallas import tpu as pltpu
from jax.experimental.pallas import tpu_sc as plsc

ENT = 100000
REL = 1000
D = 128
H = D // 2
B = 16384
LMBDA = 0.001

NC = 2
NS = 16
L = 16
NW = NC * NS
BW = B // NW
C = 64
NCHUNK = BW // C
NBUF = 2


def _pair_body(a_ref, b_ref, out_ref):
    eye = (lax.broadcasted_iota(jnp.int32, (H, H), 0)
           == lax.broadcasted_iota(jnp.int32, (H, H), 1)).astype(jnp.float32)
    dn = (((0,), (0,)), ((), ()))
    left = lax.dot_general(a_ref[...], eye, dn,
                           preferred_element_type=jnp.float32)
    right = lax.dot_general(b_ref[...], eye, dn,
                            preferred_element_type=jnp.float32)
    out_ref[...] = jnp.concatenate([left, right], axis=1)


def _pair_table(at, bt, n_rows, w):
    grid = (n_rows + w - 1) // w
    return pl.pallas_call(
        _pair_body,
        grid=(grid,),
        in_specs=[
            pl.BlockSpec((H, w), lambda i: (0, i)),
            pl.BlockSpec((H, w), lambda i: (0, i)),
        ],
        out_specs=pl.BlockSpec((w, D), lambda i: (i, 0)),
        out_shape=jax.ShapeDtypeStruct((n_rows, D), jnp.float32),
    )(at, bt)


def _sc_body(h_hbm, t_hbm, r_hbm, e12_hbm, e_hbm, r12_hbm, rel_hbm,
             res_out, ssh_out, ssd_out,
             hi_v, ti_v, ri_v, res_v, ss_v, stage_v,
             e12h_v, eh_v, e12t_v, et_v, r12_v, rel_v,
             sem0, sem1):
    wid = lax.axis_index("s") * NC + lax.axis_index("c")
    base = wid * BW

    pltpu.sync_copy(h_hbm.at[pl.ds(base, BW)], hi_v)
    pltpu.sync_copy(t_hbm.at[pl.ds(base, BW)], ti_v)
    pltpu.sync_copy(r_hbm.at[pl.ds(base, BW)], ri_v)

    sems = (sem0, sem1)

    def issue(g, slot):
        cb = g * C
        hi = hi_v.at[pl.ds(cb, C)]
        ti = ti_v.at[pl.ds(cb, C)]
        ri = ri_v.at[pl.ds(cb, C)]
        descs = []
        for tbl, idx, buf in ((e12_hbm, hi, e12h_v), (e_hbm, hi, eh_v),
                              (e12_hbm, ti, e12t_v), (e_hbm, ti, et_v),
                              (r12_hbm, ri, r12_v), (rel_hbm, ri, rel_v)):
            descs.append(pltpu.async_copy(tbl.at[idx], buf.at[slot], sems[slot]))
        return descs

    pend = [None] * NBUF
    pend[0] = issue(0, 0)
    ssh = jnp.zeros((L,), jnp.float32)
    ssd = jnp.zeros((L,), jnp.float32)
    lane = lax.iota(jnp.int32, L)

    for g in range(NCHUNK):
        slot = g % NBUF
        if g + 1 < NCHUNK:
            pend[(g + 1) % NBUF] = issue(g + 1, (g + 1) % NBUF)
        for dsc in pend[slot]:
            dsc.wait()

        def body(c, carry, slot=slot):
            ssh, ssd = carry
            acc = jnp.zeros((L,), jnp.float32)
            for k in range(H // L):
                s1 = pl.ds(k * L, L)
                s2 = pl.ds(H + k * L, L)
                a1h = e12h_v[slot, c, s1]
                a2h = e12h_v[slot, c, s2]
                a1t = e12t_v[slot, c, s1]
                a2t = e12t_v[slot, c, s2]
                v1 = r12_v[slot, c, s1]
                v2 = r12_v[slot, c, s2]
                acc = acc + (a1h * a1t + a2h * a2t) * v1 \
                          + (a1h * a2t - a2h * a1t) * v2
                ssh = ssh + (a1h * a1h + a2h * a2h) + (a1t * a1t + a2t * a2t) \
                          + (v1 * v1 + v2 * v2)
            for k in range(D // L):
                s = pl.ds(k * L, L)
                ah = eh_v[slot, c, s]
                at = et_v[slot, c, s]
                vr = rel_v[slot, c, s]
                acc = acc + ah * at * vr
                ssd = ssd + ah * ah + at * at + vr * vr
            plsc.store_scatter(stage_v, [lane * C + c], acc)
            return (ssh, ssd)

        ssh, ssd = lax.fori_loop(0, C, body, (ssh, ssd), unroll=2)

        for k in range(C // L):
            rv = stage_v[pl.ds(k * L, L)]
            for l in range(1, L):
                rv = rv + stage_v[pl.ds(l * C + k * L, L)]
            res_v[pl.ds(g * C + k * L, L)] = rv

    ss_v[0] = ssh
    ss_v[1] = ssd
    pltpu.sync_copy(res_v, res_out.at[pl.ds(base, BW)])
    pltpu.sync_copy(ss_v.at[0], ssh_out.at[pl.ds(wid * L, L)])
    pltpu.sync_copy(ss_v.at[1], ssd_out.at[pl.ds(wid * L, L)])


_sc_kernel = functools.partial(
    pl.kernel,
    out_type=(
        jax.ShapeDtypeStruct((B,), jnp.float32),
        jax.ShapeDtypeStruct((NW * L,), jnp.float32),
        jax.ShapeDtypeStruct((NW * L,), jnp.float32),
    ),
    mesh=plsc.VectorSubcoreMesh(core_axis_name="c", subcore_axis_name="s"),
    compiler_params=pltpu.CompilerParams(needs_layout_passes=False),
    scratch_types=[
        pltpu.VMEM((BW,), jnp.int32),
        pltpu.VMEM((BW,), jnp.int32),
        pltpu.VMEM((BW,), jnp.int32),
        pltpu.VMEM((BW,), jnp.float32),
        pltpu.VMEM((2, L), jnp.float32),
        pltpu.VMEM((L * C,), jnp.float32),
        pltpu.VMEM((NBUF, C, D), jnp.float32),
        pltpu.VMEM((NBUF, C, D), jnp.float32),
        pltpu.VMEM((NBUF, C, D), jnp.float32),
        pltpu.VMEM((NBUF, C, D), jnp.float32),
        pltpu.VMEM((NBUF, C, D), jnp.float32),
        pltpu.VMEM((NBUF, C, D), jnp.float32),
        pltpu.SemaphoreType.DMA,
        pltpu.SemaphoreType.DMA,
    ],
)(_sc_body)


def _finish_body(res_ref, y_ref, ssh_ref, ssd_ref, out_ref):
    x = -(y_ref[...] * res_ref[...])
    sp = jnp.maximum(x, 0.0) + jnp.log(1.0 + jnp.exp(-jnp.abs(x)))
    reg = jnp.sum(ssh_ref[...]) / (B * H) + jnp.sum(ssd_ref[...]) / (B * D)
    loss = jnp.sum(sp) / B + LMBDA * reg
    out_ref[...] = jnp.full((1, 1), loss, jnp.float32)


def kernel(h, t, r, y, E1, E2, E, R1, R2, R):
    h = h.astype(jnp.int32)
    t = t.astype(jnp.int32)
    r = r.astype(jnp.int32)
    e12 = _pair_table(E1.T, E2.T, ENT, 12800)
    r12 = _pair_table(R1.T, R2.T, REL, 1024)
    res, ssh, ssd = _sc_kernel(h, t, r, e12, E, r12, R)
    loss = pl.pallas_call(
        _finish_body,
        out_shape=jax.ShapeDtypeStruct((1, 1), jnp.float32),
    )(res.reshape(128, 128), y.reshape(128, 128),
      ssh.reshape(4, 128), ssd.reshape(4, 128))
    return loss[0, 0]
